# baseline (speedup 1.0000x reference)
import functools

import jax
import jax.numpy as jnp
from jax.experimental import pallas as pl
from jax.experimental.pallas import tpu as pltpu

N = 10000
E = 320000
HEADS = 6


def _matmul_kernel(x_ref, w_ref, o_ref):
    o_ref[...] = jnp.dot(x_ref[...], w_ref[...],
                         preferred_element_type=jnp.float32)


def _matmul(x, w, block_rows=1000):
    n, k = x.shape
    m = w.shape[1]
    return pl.pallas_call(
        _matmul_kernel,
        grid=(n // block_rows,),
        in_specs=[
            pl.BlockSpec((block_rows, k), lambda i: (i, 0)),
            pl.BlockSpec((k, m), lambda i: (0, 0)),
        ],
        out_specs=pl.BlockSpec((block_rows, m), lambda i: (i, 0)),
        out_shape=jax.ShapeDtypeStruct((n, m), jnp.float32),
    )(x, w)


def _gat_layer(x, s, d, W, att_src, att_dst, bias, out_ch):
    n = x.shape[0]
    xp = _matmul(x, W).reshape(n, HEADS, out_ch)
    a_src = (xp * att_src[None]).sum(-1)
    a_dst = (xp * att_dst[None]).sum(-1)
    alpha = a_src[s] + a_dst[d]
    alpha = jnp.where(alpha > 0, alpha, 0.2 * alpha)
    amax = jax.ops.segment_max(alpha, d, num_segments=n)
    amax = jnp.where(jnp.isfinite(amax), amax, 0.0)
    ex = jnp.exp(alpha - amax[d])
    denom = jax.ops.segment_sum(ex, d, num_segments=n)
    coef = ex / (denom[d] + 1e-16)
    msg = xp[s] * coef[:, :, None]
    out = jax.ops.segment_sum(msg, d, num_segments=n)
    return out.reshape(n, HEADS * out_ch) + bias


def kernel(x, edge_index, W1, att_src1, att_dst1, b1, W2, att_src2, att_dst2,
           b2, Wl, bl):
    src = edge_index[0]
    dst = edge_index[1]
    loop = jnp.arange(N, dtype=src.dtype)
    s = jnp.concatenate([src, loop])
    d = jnp.concatenate([dst, loop])
    h1 = jax.nn.elu(_gat_layer(x, s, d, W1, att_src1, att_dst1, b1, 30))
    h = _gat_layer(h1, s, d, W2, att_src2, att_dst2, b2, 30)
    out = _matmul(h, jnp.pad(Wl, ((0, 0), (0, 118))))[:, :10] + bl
    return (out, h)


# speedup vs baseline: 28.9643x; 28.9643x over previous
import jax
import jax.numpy as jnp
from jax import lax
from jax.experimental import pallas as pl
from jax.experimental.pallas import tpu as pltpu
from jax.experimental.pallas import tpu_sc as plsc

N = 10000
E = 320000
HEADS = 6
C = 30
HP = 32
DP = HEADS * HP
NP = 10112
STRIPE = NP // 16
CHUNK = 32
STEPS = 323
E_PAD = 32 * CHUNK * STEPS


def _matmul_kernel(x_ref, w_ref, o_ref):
    o_ref[...] = jnp.dot(x_ref[...], w_ref[...],
                         preferred_element_type=jnp.float32)


def _matmul(x, w, block_rows=1000):
    n, k = x.shape
    m = w.shape[1]
    return pl.pallas_call(
        _matmul_kernel,
        grid=(n // block_rows,),
        in_specs=[
            pl.BlockSpec((block_rows, k), lambda i: (i, 0)),
            pl.BlockSpec((k, m), lambda i: (0, 0)),
        ],
        out_specs=pl.BlockSpec((block_rows, m), lambda i: (i, 0)),
        out_shape=jax.ShapeDtypeStruct((n, m), jnp.float32),
    )(x, w)


def _edge_kernel(s_hbm, d_hbm, asrc_hbm, adst_hbm, xp_hbm,
                 numer_hbm,
                 s_ref, d_ref, a_ref, b_ref, ex_ref, x_ref,
                 numer_sh):
    c = lax.axis_index("c")
    t = lax.axis_index("s")
    w = c * 16 + t
    row0 = t * STRIPE

    lane = lax.iota(jnp.int32, 16)
    z16 = jnp.where(lane < 0, 1.0, 0.0).astype(jnp.float32)

    @pl.loop(0, CHUNK)
    def _(i):
        ex_ref[i, :] = z16
        for j in range(DP // 16):
            x_ref[i, pl.ds(16 * j, 16)] = z16

    nz = STRIPE // CHUNK
    rem = STRIPE - nz * CHUNK

    @pl.loop(0, nz)
    def _(k):
        pltpu.sync_copy(x_ref, numer_sh.at[pl.ds(row0 + CHUNK * k, CHUNK)])

    pltpu.sync_copy(x_ref.at[pl.ds(0, rem)],
                    numer_sh.at[pl.ds(row0 + nz * CHUNK, rem)])
    plsc.subcore_barrier()

    oh14 = jnp.where(lane == 14, 1.0, 0.0).astype(jnp.float32)

    base0 = w * (CHUNK * STEPS)

    @pl.loop(0, STEPS)
    def _(step):
        base = base0 + step * CHUNK
        pltpu.sync_copy(s_hbm.at[pl.ds(base, CHUNK)], s_ref)
        pltpu.sync_copy(d_hbm.at[pl.ds(base, CHUNK)], d_ref)
        pltpu.sync_copy(asrc_hbm.at[s_ref], a_ref)
        pltpu.sync_copy(adst_hbm.at[d_ref], b_ref)
        pltpu.sync_copy(xp_hbm.at[s_ref], x_ref)

        @pl.loop(0, CHUNK)
        def _(e):
            al = a_ref[e, :] + b_ref[e, :]
            ex_ref[e, :] = jnp.exp(jnp.maximum(al, 0.2 * al))

        @pl.loop(0, CHUNK)
        def _(e):
            exv = ex_ref[e, :]
            for h in range(HEADS):
                coef = exv[h]
                sl0 = pl.ds(HP * h, 16)
                sl1 = pl.ds(HP * h + 16, 16)
                x_ref[e, sl0] = x_ref[e, sl0] * coef
                x_ref[e, sl1] = (x_ref[e, sl1] + oh14) * coef

        pltpu.sync_copy(x_ref, numer_sh.at[d_ref], add=True)

    plsc.subcore_barrier()

    @pl.loop(0, nz)
    def _(k):
        pltpu.sync_copy(numer_sh.at[pl.ds(row0 + CHUNK * k, CHUNK)],
                        numer_hbm.at[c, pl.ds(row0 + CHUNK * k, CHUNK)])

    pltpu.sync_copy(numer_sh.at[pl.ds(row0 + nz * CHUNK, rem)],
                    numer_hbm.at[c, pl.ds(row0 + nz * CHUNK, rem)])


_EDGE_CALL = pl.kernel(
    _edge_kernel,
    out_type=jax.ShapeDtypeStruct((2, NP, DP), jnp.float32),
    mesh=plsc.VectorSubcoreMesh(core_axis_name="c", subcore_axis_name="s"),
    scratch_types=[
        pltpu.VMEM((CHUNK,), jnp.int32),
        pltpu.VMEM((CHUNK,), jnp.int32),
        pltpu.VMEM((CHUNK, 16), jnp.float32),
        pltpu.VMEM((CHUNK, 16), jnp.float32),
        pltpu.VMEM((CHUNK, 16), jnp.float32),
        pltpu.VMEM((CHUNK, DP), jnp.float32),
        pltpu.VMEM_SHARED((NP, DP), jnp.float32),
    ],
    compiler_params=pltpu.CompilerParams(use_tc_tiling_on_sc=False),
)


def _ext_weight(W, att_src, att_dst):
    fin = W.shape[0]
    Wr = W.reshape(fin, HEADS, C)
    vs = jnp.einsum("fhc,hc->fh", Wr, att_src)
    vd = jnp.einsum("fhc,hc->fh", Wr, att_dst)
    return jnp.concatenate([W, vs, vd], axis=1)


def _gat_layer(xw, s_idx, d_idx, W_ext, bias):
    mm = _matmul(xw, W_ext)
    xp = mm[:, :HEADS * C].reshape(N, HEADS, C)
    xp_pad = jnp.pad(xp, ((0, NP - N), (0, 0), (0, HP - C))).reshape(NP, DP)
    asrc = jnp.pad(mm[:, 180:186], ((0, NP - N), (0, 10)))
    adst = jnp.pad(mm[:, 186:192], ((0, NP - N), (0, 10)))
    numer = _EDGE_CALL(s_idx, d_idx, asrc, adst, xp_pad)
    acc = (numer[0, :N] + numer[1, :N]).reshape(N, HEADS, HP)
    out = acc[:, :, :C] / (acc[:, :, C:C + 1] + 1e-16)
    return out.reshape(N, HEADS * C) + bias


def kernel(x, edge_index, W1, att_src1, att_dst1, b1, W2, att_src2, att_dst2,
           b2, Wl, bl):
    src = edge_index[0]
    dst = edge_index[1]
    loop = jnp.arange(N, dtype=src.dtype)
    pad = jnp.full((E_PAD - E - N,), N, src.dtype)
    s_idx = jnp.concatenate([src, loop, pad])
    d_idx = jnp.concatenate([dst, loop, pad])

    W1e = _ext_weight(W1, att_src1, att_dst1)
    h1 = jax.nn.elu(_gat_layer(x, s_idx, d_idx, W1e, b1))

    W2e = jnp.pad(_ext_weight(W2, att_src2, att_dst2), ((0, 12), (0, 0)))
    h1p = jnp.pad(h1, ((0, 0), (0, 12)))
    h = _gat_layer(h1p, s_idx, d_idx, W2e, b2)

    out = _matmul(jnp.pad(h, ((0, 0), (0, 12))),
                  jnp.pad(Wl, ((0, 12), (0, 118))))[:, :10] + bl
    return (out, h)


# speedup vs baseline: 32.1635x; 1.1105x over previous
import jax
import jax.numpy as jnp
from jax import lax
from jax.experimental import pallas as pl
from jax.experimental.pallas import tpu as pltpu
from jax.experimental.pallas import tpu_sc as plsc

N = 10000
E = 320000
HEADS = 6
C = 30
HP = 32
DP = HEADS * HP
NP = 10112
STRIPE = NP // 16
CHUNK = 32
STEPS = 323
E_PAD = 32 * CHUNK * STEPS


def _matmul_kernel(x_ref, w_ref, o_ref):
    o_ref[...] = jnp.dot(x_ref[...], w_ref[...],
                         preferred_element_type=jnp.float32)


def _matmul(x, w, block_rows=1000):
    n, k = x.shape
    m = w.shape[1]
    return pl.pallas_call(
        _matmul_kernel,
        grid=(n // block_rows,),
        in_specs=[
            pl.BlockSpec((block_rows, k), lambda i: (i, 0)),
            pl.BlockSpec((k, m), lambda i: (0, 0)),
        ],
        out_specs=pl.BlockSpec((block_rows, m), lambda i: (i, 0)),
        out_shape=jax.ShapeDtypeStruct((n, m), jnp.float32),
    )(x, w)


def _edge_kernel(sd_hbm, adst_hbm, xp_hbm,
                 numer_hbm,
                 sd_ref, b_ref, x_ref,
                 numer_sh):
    c = lax.axis_index("c")
    t = lax.axis_index("s")
    w = c * 16 + t
    row0 = t * STRIPE

    lane = lax.iota(jnp.int32, 16)
    z16 = jnp.where(lane < 0, 1.0, 0.0).astype(jnp.float32)
    oh = [jnp.where(lane == i, 1.0, 0.0).astype(jnp.float32)
          for i in range(HEADS)]
    oh14 = jnp.where(lane == 14, 1.0, 0.0).astype(jnp.float32)

    @pl.loop(0, CHUNK)
    def _(i):
        for j in range(DP // 16):
            x_ref[i, pl.ds(16 * j, 16)] = z16

    nz = STRIPE // CHUNK
    rem = STRIPE - nz * CHUNK

    @pl.loop(0, nz)
    def _(k):
        pltpu.sync_copy(x_ref, numer_sh.at[pl.ds(row0 + CHUNK * k, CHUNK)])

    pltpu.sync_copy(x_ref.at[pl.ds(0, rem)],
                    numer_sh.at[pl.ds(row0 + nz * CHUNK, rem)])
    plsc.subcore_barrier()

    base0 = w * STEPS

    @pl.loop(0, STEPS)
    def _(step):
        gstep = base0 + step
        pltpu.sync_copy(sd_hbm.at[pl.ds(gstep * 2 * CHUNK, 2 * CHUNK)],
                        sd_ref)
        s_sl = sd_ref.at[pl.ds(0, CHUNK)]
        d_sl = sd_ref.at[pl.ds(CHUNK, CHUNK)]
        pltpu.sync_copy(adst_hbm.at[d_sl], b_ref)
        pltpu.sync_copy(xp_hbm.at[s_sl], x_ref)

        @pl.loop(0, CHUNK)
        def _(e):
            x1 = [x_ref[e, pl.ds(HP * h + 16, 16)] for h in range(HEADS)]
            al = b_ref[e, :]
            for h in range(HEADS):
                al = al + x1[h][15] * oh[h]
            exv = jnp.exp(jnp.maximum(al, 0.2 * al))
            for h in range(HEADS):
                coef = exv[h]
                sl0 = pl.ds(HP * h, 16)
                x_ref[e, sl0] = x_ref[e, sl0] * coef
                x_ref[e, pl.ds(HP * h + 16, 16)] = (x1[h] + oh14) * coef

        pltpu.sync_copy(x_ref, numer_sh.at[d_sl], add=True)

    plsc.subcore_barrier()

    @pl.loop(0, nz)
    def _(k):
        pltpu.sync_copy(numer_sh.at[pl.ds(row0 + CHUNK * k, CHUNK)],
                        numer_hbm.at[c, pl.ds(row0 + CHUNK * k, CHUNK)])

    pltpu.sync_copy(numer_sh.at[pl.ds(row0 + nz * CHUNK, rem)],
                    numer_hbm.at[c, pl.ds(row0 + nz * CHUNK, rem)])


_EDGE_CALL = pl.kernel(
    _edge_kernel,
    out_type=jax.ShapeDtypeStruct((2, NP, DP), jnp.float32),
    mesh=plsc.VectorSubcoreMesh(core_axis_name="c", subcore_axis_name="s"),
    scratch_types=[
        pltpu.VMEM((2 * CHUNK,), jnp.int32),
        pltpu.VMEM((CHUNK, 16), jnp.float32),
        pltpu.VMEM((CHUNK, DP), jnp.float32),
        pltpu.VMEM_SHARED((NP, DP), jnp.float32),
    ],
    compiler_params=pltpu.CompilerParams(use_tc_tiling_on_sc=False),
)


def _ext_weight(W, att_src, att_dst):
    fin = W.shape[0]
    Wr = W.reshape(fin, HEADS, C)
    vs = jnp.einsum("fhc,hc->fh", Wr, att_src)
    vd = jnp.einsum("fhc,hc->fh", Wr, att_dst)
    return jnp.concatenate([W, vs, vd], axis=1)


def _gat_layer(xw, sd_idx, W_ext, bias):
    mm = _matmul(xw, W_ext)
    xp = mm[:, :HEADS * C].reshape(N, HEADS, C)
    xp_row = jnp.concatenate(
        [xp, jnp.zeros((N, HEADS, 1), jnp.float32),
         mm[:, 180:186].reshape(N, HEADS, 1)], axis=2)
    xp_pad = jnp.pad(xp_row.reshape(N, DP), ((0, NP - N), (0, 0)))
    adst = jnp.pad(mm[:, 186:192], ((0, NP - N), (0, 10)))
    numer = _EDGE_CALL(sd_idx, adst, xp_pad)
    acc = (numer[0, :N] + numer[1, :N]).reshape(N, HEADS, HP)
    out = acc[:, :, :C] / (acc[:, :, C:C + 1] + 1e-16)
    return out.reshape(N, HEADS * C) + bias


def kernel(x, edge_index, W1, att_src1, att_dst1, b1, W2, att_src2, att_dst2,
           b2, Wl, bl):
    src = edge_index[0]
    dst = edge_index[1]
    loop = jnp.arange(N, dtype=src.dtype)
    pad = jnp.full((E_PAD - E - N,), N, src.dtype)
    s_idx = jnp.concatenate([src, loop, pad]).reshape(-1, CHUNK)
    d_idx = jnp.concatenate([dst, loop, pad]).reshape(-1, CHUNK)
    sd_idx = jnp.stack([s_idx, d_idx], axis=1).reshape(-1)

    W1e = _ext_weight(W1, att_src1, att_dst1)
    h1 = jax.nn.elu(_gat_layer(x, sd_idx, W1e, b1))

    W2e = jnp.pad(_ext_weight(W2, att_src2, att_dst2), ((0, 12), (0, 0)))
    h1p = jnp.pad(h1, ((0, 0), (0, 12)))
    h = _gat_layer(h1p, sd_idx, W2e, b2)

    out = _matmul(jnp.pad(h, ((0, 0), (0, 12))),
                  jnp.pad(Wl, ((0, 12), (0, 118))))[:, :10] + bl
    return (out, h)


# speedup vs baseline: 48.6008x; 1.5111x over previous
import jax
import jax.numpy as jnp
from jax import lax
from jax.experimental import pallas as pl
from jax.experimental.pallas import tpu as pltpu
from jax.experimental.pallas import tpu_sc as plsc

N = 10000
E = 320000
HEADS = 6
C = 30
HP = 32
DP = HEADS * HP
NP = 10112
STRIPE = NP // 16
CHUNK = 16
STEPS = 648
E_PAD = 32 * CHUNK * STEPS


def _matmul_kernel(x_ref, w_ref, o_ref):
    o_ref[...] = jnp.dot(x_ref[...], w_ref[...],
                         preferred_element_type=jnp.float32)


def _matmul(x, w, block_rows=1000):
    n, k = x.shape
    m = w.shape[1]
    return pl.pallas_call(
        _matmul_kernel,
        grid=(n // block_rows,),
        in_specs=[
            pl.BlockSpec((block_rows, k), lambda i: (i, 0)),
            pl.BlockSpec((k, m), lambda i: (0, 0)),
        ],
        out_specs=pl.BlockSpec((block_rows, m), lambda i: (i, 0)),
        out_shape=jax.ShapeDtypeStruct((n, m), jnp.float32),
    )(x, w)


def _edge_kernel(sd_hbm, adst_hbm, xp_hbm,
                 numer_hbm,
                 sd0, sd1, sd2, b0, b1, x0, x1,
                 sdm0, sdm1, sdm2, gm0, gm1, sc0, sc1,
                 numer_sh):
    c = lax.axis_index("c")
    t = lax.axis_index("s")
    w = c * 16 + t
    row0 = t * STRIPE

    sd = [sd0, sd1, sd2]
    sdm = [sdm0, sdm1, sdm2]
    bb = [b0, b1]
    xx = [x0, x1]
    gm = [gm0, gm1]
    sc = [sc0, sc1]

    lane = lax.iota(jnp.int32, 16)
    z16 = jnp.where(lane < 0, 1.0, 0.0).astype(jnp.float32)
    oh = [jnp.where(lane == i, 1.0, 0.0).astype(jnp.float32)
          for i in range(HEADS)]
    oh14 = jnp.where(lane == 14, 1.0, 0.0).astype(jnp.float32)

    @pl.loop(0, CHUNK)
    def _(i):
        for j in range(DP // 16):
            x0[i, pl.ds(16 * j, 16)] = z16

    nz = STRIPE // CHUNK
    rem = STRIPE - nz * CHUNK

    @pl.loop(0, nz)
    def _(k):
        pltpu.sync_copy(x0, numer_sh.at[pl.ds(row0 + CHUNK * k, CHUNK)])

    pltpu.sync_copy(x0.at[pl.ds(0, rem)],
                    numer_sh.at[pl.ds(row0 + nz * CHUNK, rem)])
    plsc.subcore_barrier()

    base0 = w * STEPS

    def sd_src(g):
        return sd_hbm.at[pl.ds((base0 + g) * 2 * CHUNK, 2 * CHUNK)]

    def issue_sd(g, p):
        pltpu.async_copy(sd_src(g), sd[p], sdm[p])

    def wait_sd(g, p):
        pltpu.make_async_copy(sd_src(g), sd[p], sdm[p]).wait()

    def issue_gather(p, q):
        s_sl = sd[p].at[pl.ds(0, CHUNK)]
        d_sl = sd[p].at[pl.ds(CHUNK, CHUNK)]
        pltpu.async_copy(adst_hbm.at[d_sl], bb[q], gm[q])
        pltpu.async_copy(xp_hbm.at[s_sl], xx[q], gm[q])

    def wait_gather(p, q):
        d_sl = sd[p].at[pl.ds(CHUNK, CHUNK)]
        s_sl = sd[p].at[pl.ds(0, CHUNK)]
        pltpu.make_async_copy(adst_hbm.at[d_sl], bb[q], gm[q]).wait()
        pltpu.make_async_copy(xp_hbm.at[s_sl], xx[q], gm[q]).wait()

    def issue_scatter(p, q):
        d_sl = sd[p].at[pl.ds(CHUNK, CHUNK)]
        pltpu.async_copy(xx[q], numer_sh.at[d_sl], sc[q], add=True)

    def wait_scatter(p, q):
        d_sl = sd[p].at[pl.ds(CHUNK, CHUNK)]
        pltpu.make_async_copy(xx[q], numer_sh.at[d_sl], sc[q]).wait()

    def compute(q):
        b_ref = bb[q]
        x_ref = xx[q]

        @pl.loop(0, CHUNK)
        def _(e):
            x1v = [x_ref[e, pl.ds(HP * h + 16, 16)] for h in range(HEADS)]
            al = b_ref[e, :]
            for h in range(HEADS):
                al = al + x1v[h][15] * oh[h]
            exv = jnp.exp(jnp.maximum(al, 0.2 * al))
            for h in range(HEADS):
                coef = exv[h]
                sl0 = pl.ds(HP * h, 16)
                x_ref[e, sl0] = x_ref[e, sl0] * coef
                x_ref[e, pl.ds(HP * h + 16, 16)] = (x1v[h] + oh14) * coef

    issue_sd(0, 0)
    issue_sd(1, 1)
    wait_sd(0, 0)
    issue_gather(0, 0)

    @pl.loop(0, STEPS // 6)
    def _(i):
        g0 = 6 * i
        for k in range(6):
            g = g0 + k
            p = k % 3
            pn = (k + 1) % 3
            pf = (k + 2) % 3
            q = k % 2
            qo = 1 - q

            @pl.when(g >= 1)
            def _():
                wait_scatter((k - 1) % 3, qo)

            @pl.when(g + 2 < STEPS)
            def _():
                issue_sd(g + 2, pf)

            @pl.when(g + 1 < STEPS)
            def _():
                wait_sd(g + 1, pn)
                issue_gather(pn, qo)

            wait_gather(p, q)
            compute(q)
            issue_scatter(p, q)

    wait_scatter((STEPS - 1) % 3, (STEPS - 1) % 2)
    plsc.subcore_barrier()

    @pl.loop(0, nz)
    def _(k):
        pltpu.sync_copy(numer_sh.at[pl.ds(row0 + CHUNK * k, CHUNK)],
                        numer_hbm.at[c, pl.ds(row0 + CHUNK * k, CHUNK)])

    pltpu.sync_copy(numer_sh.at[pl.ds(row0 + nz * CHUNK, rem)],
                    numer_hbm.at[c, pl.ds(row0 + nz * CHUNK, rem)])


_EDGE_CALL = pl.kernel(
    _edge_kernel,
    out_type=jax.ShapeDtypeStruct((2, NP, DP), jnp.float32),
    mesh=plsc.VectorSubcoreMesh(core_axis_name="c", subcore_axis_name="s"),
    scratch_types=[
        pltpu.VMEM((2 * CHUNK,), jnp.int32),
        pltpu.VMEM((2 * CHUNK,), jnp.int32),
        pltpu.VMEM((2 * CHUNK,), jnp.int32),
        pltpu.VMEM((CHUNK, 16), jnp.float32),
        pltpu.VMEM((CHUNK, 16), jnp.float32),
        pltpu.VMEM((CHUNK, DP), jnp.float32),
        pltpu.VMEM((CHUNK, DP), jnp.float32),
        pltpu.SemaphoreType.DMA,
        pltpu.SemaphoreType.DMA,
        pltpu.SemaphoreType.DMA,
        pltpu.SemaphoreType.DMA,
        pltpu.SemaphoreType.DMA,
        pltpu.SemaphoreType.DMA,
        pltpu.SemaphoreType.DMA,
        pltpu.VMEM_SHARED((NP, DP), jnp.float32),
    ],
    compiler_params=pltpu.CompilerParams(use_tc_tiling_on_sc=False),
)


def _ext_weight(W, att_src, att_dst):
    fin = W.shape[0]
    Wr = W.reshape(fin, HEADS, C)
    vs = jnp.einsum("fhc,hc->fh", Wr, att_src)
    vd = jnp.einsum("fhc,hc->fh", Wr, att_dst)
    return jnp.concatenate([W, vs, vd], axis=1)


def _gat_layer(xw, sd_idx, W_ext, bias):
    mm = _matmul(xw, W_ext)
    xp = mm[:, :HEADS * C].reshape(N, HEADS, C)
    xp_row = jnp.concatenate(
        [xp, jnp.zeros((N, HEADS, 1), jnp.float32),
         mm[:, 180:186].reshape(N, HEADS, 1)], axis=2)
    xp_pad = jnp.pad(xp_row.reshape(N, DP), ((0, NP - N), (0, 0)))
    adst = jnp.pad(mm[:, 186:192], ((0, NP - N), (0, 10)))
    numer = _EDGE_CALL(sd_idx, adst, xp_pad)
    acc = (numer[0, :N] + numer[1, :N]).reshape(N, HEADS, HP)
    out = acc[:, :, :C] / (acc[:, :, C:C + 1] + 1e-16)
    return out.reshape(N, HEADS * C) + bias


def kernel(x, edge_index, W1, att_src1, att_dst1, b1, W2, att_src2, att_dst2,
           b2, Wl, bl):
    src = edge_index[0]
    dst = edge_index[1]
    loop = jnp.arange(N, dtype=src.dtype)
    pad = jnp.full((E_PAD - E - N,), N, src.dtype)
    s_idx = jnp.concatenate([src, loop, pad]).reshape(-1, CHUNK)
    d_idx = jnp.concatenate([dst, loop, pad]).reshape(-1, CHUNK)
    sd_idx = jnp.stack([s_idx, d_idx], axis=1).reshape(-1)

    W1e = _ext_weight(W1, att_src1, att_dst1)
    h1 = jax.nn.elu(_gat_layer(x, sd_idx, W1e, b1))

    W2e = jnp.pad(_ext_weight(W2, att_src2, att_dst2), ((0, 12), (0, 0)))
    h1p = jnp.pad(h1, ((0, 0), (0, 12)))
    h = _gat_layer(h1p, sd_idx, W2e, b2)

    out = _matmul(jnp.pad(h, ((0, 0), (0, 12))),
                  jnp.pad(Wl, ((0, 12), (0, 118))))[:, :10] + bl
    return (out, h)


# speedup vs baseline: 52.9852x; 1.0902x over previous
import jax
import jax.numpy as jnp
from jax import lax
from jax.experimental import pallas as pl
from jax.experimental.pallas import tpu as pltpu
from jax.experimental.pallas import tpu_sc as plsc

N = 10000
E = 320000
HEADS = 6
C = 30
HP = 32
DP = HEADS * HP
NP = 10112
STRIPE = NP // 16
CHUNK = 16
STEPS = 648
E_PAD = 32 * CHUNK * STEPS


def _matmul_kernel(x_ref, w_ref, o_ref):
    o_ref[...] = jnp.dot(x_ref[...], w_ref[...],
                         preferred_element_type=jnp.float32)


def _proj_kernel(x_ref, w_ref, o1_ref, o2_ref):
    r = jnp.dot(x_ref[...], w_ref[...], preferred_element_type=jnp.float32)
    o1_ref[...] = r[:, :DP]
    o2_ref[...] = r[:, DP:]


def _proj(x, w, block_rows=1000):
    n, k = x.shape
    return pl.pallas_call(
        _proj_kernel,
        grid=(n // block_rows,),
        in_specs=[
            pl.BlockSpec((block_rows, k), lambda i: (i, 0)),
            pl.BlockSpec((k, DP + 16), lambda i: (0, 0)),
        ],
        out_specs=[pl.BlockSpec((block_rows, DP), lambda i: (i, 0)),
                   pl.BlockSpec((block_rows, 16), lambda i: (i, 0))],
        out_shape=[jax.ShapeDtypeStruct((NP, DP), jnp.float32),
                   jax.ShapeDtypeStruct((NP, 16), jnp.float32)],
    )(x, w)


def _matmul(x, w, block_rows=1000):
    n, k = x.shape
    m = w.shape[1]
    return pl.pallas_call(
        _matmul_kernel,
        grid=(n // block_rows,),
        in_specs=[
            pl.BlockSpec((block_rows, k), lambda i: (i, 0)),
            pl.BlockSpec((k, m), lambda i: (0, 0)),
        ],
        out_specs=pl.BlockSpec((block_rows, m), lambda i: (i, 0)),
        out_shape=jax.ShapeDtypeStruct((n, m), jnp.float32),
    )(x, w)


def _edge_kernel(sd_hbm, adst_hbm, xp_hbm,
                 numer_hbm,
                 sd0, sd1, sd2, b0, b1, x0, x1,
                 sdm0, sdm1, sdm2, gm0, gm1, sc0, sc1,
                 numer_sh):
    c = lax.axis_index("c")
    t = lax.axis_index("s")
    w = c * 16 + t
    row0 = t * STRIPE

    sd = [sd0, sd1, sd2]
    sdm = [sdm0, sdm1, sdm2]
    bb = [b0, b1]
    xx = [x0, x1]
    gm = [gm0, gm1]
    sc = [sc0, sc1]

    lane = lax.iota(jnp.int32, 16)
    z16 = jnp.where(lane < 0, 1.0, 0.0).astype(jnp.float32)
    oh = [jnp.where(lane == i, 1.0, 0.0).astype(jnp.float32)
          for i in range(HEADS)]
    oh14 = jnp.where(lane == 14, 1.0, 0.0).astype(jnp.float32)

    @pl.loop(0, CHUNK)
    def _(i):
        for j in range(DP // 16):
            x0[i, pl.ds(16 * j, 16)] = z16

    nz = STRIPE // CHUNK
    rem = STRIPE - nz * CHUNK

    @pl.loop(0, nz)
    def _(k):
        pltpu.sync_copy(x0, numer_sh.at[pl.ds(row0 + CHUNK * k, CHUNK)])

    pltpu.sync_copy(x0.at[pl.ds(0, rem)],
                    numer_sh.at[pl.ds(row0 + nz * CHUNK, rem)])
    plsc.subcore_barrier()

    base0 = w * STEPS

    def sd_src(g):
        return sd_hbm.at[pl.ds((base0 + g) * 2 * CHUNK, 2 * CHUNK)]

    def issue_sd(g, p):
        pltpu.async_copy(sd_src(g), sd[p], sdm[p])

    def wait_sd(g, p):
        pltpu.make_async_copy(sd_src(g), sd[p], sdm[p]).wait()

    def issue_gather(p, q):
        s_sl = sd[p].at[pl.ds(0, CHUNK)]
        d_sl = sd[p].at[pl.ds(CHUNK, CHUNK)]
        pltpu.async_copy(adst_hbm.at[d_sl], bb[q], gm[q])
        pltpu.async_copy(xp_hbm.at[s_sl], xx[q], gm[q])

    def wait_gather(p, q):
        d_sl = sd[p].at[pl.ds(CHUNK, CHUNK)]
        s_sl = sd[p].at[pl.ds(0, CHUNK)]
        pltpu.make_async_copy(adst_hbm.at[d_sl], bb[q], gm[q]).wait()
        pltpu.make_async_copy(xp_hbm.at[s_sl], xx[q], gm[q]).wait()

    def issue_scatter(p, q):
        d_sl = sd[p].at[pl.ds(CHUNK, CHUNK)]
        pltpu.async_copy(xx[q], numer_sh.at[d_sl], sc[q], add=True)

    def wait_scatter(p, q):
        d_sl = sd[p].at[pl.ds(CHUNK, CHUNK)]
        pltpu.make_async_copy(xx[q], numer_sh.at[d_sl], sc[q]).wait()

    def compute(q):
        b_ref = bb[q]
        x_ref = xx[q]

        @pl.loop(0, CHUNK, step=2)
        def _(e0):
            for u in range(2):
                e = e0 + u
                x1v = [x_ref[e, pl.ds(HP * h + 16, 16)] for h in range(HEADS)]
                al = b_ref[e, :]
                for h in range(HEADS):
                    al = al + x1v[h][15] * oh[h]
                exv = jnp.exp(jnp.maximum(al, 0.2 * al))
                for h in range(HEADS):
                    coef = exv[h]
                    sl0 = pl.ds(HP * h, 16)
                    x_ref[e, sl0] = x_ref[e, sl0] * coef
                    x_ref[e, pl.ds(HP * h + 16, 16)] = (x1v[h] + oh14) * coef

    issue_sd(0, 0)
    issue_sd(1, 1)
    wait_sd(0, 0)
    issue_gather(0, 0)

    @pl.loop(0, STEPS // 6)
    def _(i):
        g0 = 6 * i
        for k in range(6):
            g = g0 + k
            p = k % 3
            pn = (k + 1) % 3
            pf = (k + 2) % 3
            q = k % 2
            qo = 1 - q

            @pl.when(g >= 1)
            def _():
                wait_scatter((k - 1) % 3, qo)

            @pl.when(g + 2 < STEPS)
            def _():
                issue_sd(g + 2, pf)

            @pl.when(g + 1 < STEPS)
            def _():
                wait_sd(g + 1, pn)
                issue_gather(pn, qo)

            wait_gather(p, q)
            compute(q)
            issue_scatter(p, q)

    wait_scatter((STEPS - 1) % 3, (STEPS - 1) % 2)
    plsc.subcore_barrier()

    @pl.loop(0, nz)
    def _(k):
        pltpu.sync_copy(numer_sh.at[pl.ds(row0 + CHUNK * k, CHUNK)],
                        numer_hbm.at[c, pl.ds(row0 + CHUNK * k, CHUNK)])

    pltpu.sync_copy(numer_sh.at[pl.ds(row0 + nz * CHUNK, rem)],
                    numer_hbm.at[c, pl.ds(row0 + nz * CHUNK, rem)])


_EDGE_CALL = pl.kernel(
    _edge_kernel,
    out_type=jax.ShapeDtypeStruct((2, NP, DP), jnp.float32),
    mesh=plsc.VectorSubcoreMesh(core_axis_name="c", subcore_axis_name="s"),
    scratch_types=[
        pltpu.VMEM((2 * CHUNK,), jnp.int32),
        pltpu.VMEM((2 * CHUNK,), jnp.int32),
        pltpu.VMEM((2 * CHUNK,), jnp.int32),
        pltpu.VMEM((CHUNK, 16), jnp.float32),
        pltpu.VMEM((CHUNK, 16), jnp.float32),
        pltpu.VMEM((CHUNK, DP), jnp.float32),
        pltpu.VMEM((CHUNK, DP), jnp.float32),
        pltpu.SemaphoreType.DMA,
        pltpu.SemaphoreType.DMA,
        pltpu.SemaphoreType.DMA,
        pltpu.SemaphoreType.DMA,
        pltpu.SemaphoreType.DMA,
        pltpu.SemaphoreType.DMA,
        pltpu.SemaphoreType.DMA,
        pltpu.VMEM_SHARED((NP, DP), jnp.float32),
    ],
    compiler_params=pltpu.CompilerParams(use_tc_tiling_on_sc=False),
)


def _ext_weight(W, att_src, att_dst):
    fin = W.shape[0]
    Wr = W.reshape(fin, HEADS, C)
    vs = jnp.einsum("fhc,hc->fh", Wr, att_src)
    vd = jnp.einsum("fhc,hc->fh", Wr, att_dst)
    Wh = jnp.concatenate(
        [Wr, jnp.zeros((fin, HEADS, 1), jnp.float32), vs[:, :, None]],
        axis=2).reshape(fin, DP)
    return jnp.concatenate(
        [Wh, vd, jnp.zeros((fin, 10), jnp.float32)], axis=1)


def _gat_layer(xw, sd_idx, W_ext, bias):
    xp_pad, adst = _proj(xw, W_ext)
    numer = _EDGE_CALL(sd_idx, adst, xp_pad)
    acc = (numer[0, :N] + numer[1, :N]).reshape(N, HEADS, HP)
    out = acc[:, :, :C] / (acc[:, :, C:C + 1] + 1e-16)
    return out.reshape(N, HEADS * C) + bias


def kernel(x, edge_index, W1, att_src1, att_dst1, b1, W2, att_src2, att_dst2,
           b2, Wl, bl):
    src = edge_index[0]
    dst = edge_index[1]
    loop = jnp.arange(N, dtype=src.dtype)
    pad = jnp.full((E_PAD - E - N,), N, src.dtype)
    s_idx = jnp.concatenate([src, loop, pad]).reshape(-1, CHUNK)
    d_idx = jnp.concatenate([dst, loop, pad]).reshape(-1, CHUNK)
    sd_idx = jnp.stack([s_idx, d_idx], axis=1).reshape(-1)

    W1e = _ext_weight(W1, att_src1, att_dst1)
    h1 = jax.nn.elu(_gat_layer(x, sd_idx, W1e, b1))

    W2e = jnp.pad(_ext_weight(W2, att_src2, att_dst2), ((0, 12), (0, 0)))
    h1p = jnp.pad(h1, ((0, 0), (0, 12)))
    h = _gat_layer(h1p, sd_idx, W2e, b2)

    out = _matmul(jnp.pad(h, ((0, 0), (0, 12))),
                  jnp.pad(Wl, ((0, 12), (0, 118))))[:, :10] + bl
    return (out, h)


# speedup vs baseline: 63.3238x; 1.1951x over previous
import jax
import jax.numpy as jnp
from jax import lax
from jax.experimental import pallas as pl
from jax.experimental.pallas import tpu as pltpu
from jax.experimental.pallas import tpu_sc as plsc

N = 10000
E = 320000
HEADS = 6
C = 30
HP = 32
DP = HEADS * HP
NP = 10112
STRIPE = NP // 16
CHUNK = 16
STEPS = 648
E_PAD = 32 * CHUNK * STEPS


def _matmul_kernel(x_ref, w_ref, o_ref):
    o_ref[...] = jnp.dot(x_ref[...], w_ref[...],
                         preferred_element_type=jnp.float32)


def _proj_kernel(x_ref, w_ref, o1_ref, o2_ref):
    r = jnp.dot(x_ref[...], w_ref[...], preferred_element_type=jnp.float32)
    o1_ref[...] = r[:, :DP]
    o2_ref[...] = r[:, DP:]


def _proj(x, w, block_rows=1000):
    n, k = x.shape
    return pl.pallas_call(
        _proj_kernel,
        grid=(n // block_rows,),
        in_specs=[
            pl.BlockSpec((block_rows, k), lambda i: (i, 0)),
            pl.BlockSpec((k, DP + 16), lambda i: (0, 0)),
        ],
        out_specs=[pl.BlockSpec((block_rows, DP), lambda i: (i, 0)),
                   pl.BlockSpec((block_rows, 16), lambda i: (i, 0))],
        out_shape=[jax.ShapeDtypeStruct((NP, DP), jnp.float32),
                   jax.ShapeDtypeStruct((NP, 16), jnp.float32)],
    )(x, w)


def _epi_kernel_elu(nu_ref, b_ref, o_ref):
    _epi_body(nu_ref, b_ref, o_ref, True)


def _epi_kernel_lin(nu_ref, b_ref, o_ref):
    _epi_body(nu_ref, b_ref, o_ref, False)


def _epi_body(nu_ref, b_ref, o_ref, elu):
    acc = nu_ref[0] + nu_ref[1]
    for h in range(HEADS):
        a = acc[:, HP * h:HP * h + C]
        den = acc[:, HP * h + C:HP * h + C + 1]
        v = a / (den + 1e-16) + b_ref[:, HP * h:HP * h + C]
        if elu:
            v = jnp.where(v > 0, v, jnp.exp(v) - 1.0)
        o_ref[:, HP * h:HP * h + C] = v
        o_ref[:, HP * h + C:HP * h + HP] = jnp.zeros(
            (o_ref.shape[0], HP - C), jnp.float32)


def _epilogue(numer, bias_pad, elu, block_rows=1000):
    body = _epi_kernel_elu if elu else _epi_kernel_lin
    return pl.pallas_call(
        body,
        grid=(N // block_rows,),
        in_specs=[
            pl.BlockSpec((2, block_rows, DP), lambda i: (0, i, 0)),
            pl.BlockSpec((1, DP), lambda i: (0, 0)),
        ],
        out_specs=pl.BlockSpec((block_rows, DP), lambda i: (i, 0)),
        out_shape=jax.ShapeDtypeStruct((N, DP), jnp.float32),
    )(numer, bias_pad)


def _pad_heads(a):
    return jnp.pad(a.reshape(HEADS, C, -1),
                   ((0, 0), (0, HP - C), (0, 0))).reshape(HEADS * HP, -1)


def _matmul(x, w, block_rows=1000):
    n, k = x.shape
    m = w.shape[1]
    return pl.pallas_call(
        _matmul_kernel,
        grid=(n // block_rows,),
        in_specs=[
            pl.BlockSpec((block_rows, k), lambda i: (i, 0)),
            pl.BlockSpec((k, m), lambda i: (0, 0)),
        ],
        out_specs=pl.BlockSpec((block_rows, m), lambda i: (i, 0)),
        out_shape=jax.ShapeDtypeStruct((n, m), jnp.float32),
    )(x, w)


def _edge_kernel(sd_hbm, adst_hbm, xp_hbm,
                 numer_hbm,
                 sd0, sd1, sd2, b0, b1, x0, x1,
                 sdm0, sdm1, sdm2, gm0, gm1, sc0, sc1,
                 numer_sh):
    c = lax.axis_index("c")
    t = lax.axis_index("s")
    w = c * 16 + t
    row0 = t * STRIPE

    sd = [sd0, sd1, sd2]
    sdm = [sdm0, sdm1, sdm2]
    bb = [b0, b1]
    xx = [x0, x1]
    gm = [gm0, gm1]
    sc = [sc0, sc1]

    lane = lax.iota(jnp.int32, 16)
    z16 = jnp.where(lane < 0, 1.0, 0.0).astype(jnp.float32)
    oh = [jnp.where(lane == i, 1.0, 0.0).astype(jnp.float32)
          for i in range(HEADS)]
    oh14 = jnp.where(lane == 14, 1.0, 0.0).astype(jnp.float32)

    @pl.loop(0, CHUNK)
    def _(i):
        for j in range(DP // 16):
            x0[i, pl.ds(16 * j, 16)] = z16

    nz = STRIPE // CHUNK
    rem = STRIPE - nz * CHUNK

    @pl.loop(0, nz)
    def _(k):
        pltpu.sync_copy(x0, numer_sh.at[pl.ds(row0 + CHUNK * k, CHUNK)])

    pltpu.sync_copy(x0.at[pl.ds(0, rem)],
                    numer_sh.at[pl.ds(row0 + nz * CHUNK, rem)])
    plsc.subcore_barrier()

    base0 = w * STEPS

    def sd_src(g):
        return sd_hbm.at[pl.ds((base0 + g) * 2 * CHUNK, 2 * CHUNK)]

    def issue_sd(g, p):
        pltpu.async_copy(sd_src(g), sd[p], sdm[p])

    def wait_sd(g, p):
        pltpu.make_async_copy(sd_src(g), sd[p], sdm[p]).wait()

    def issue_gather(p, q):
        s_sl = sd[p].at[pl.ds(0, CHUNK)]
        d_sl = sd[p].at[pl.ds(CHUNK, CHUNK)]
        pltpu.async_copy(adst_hbm.at[d_sl], bb[q], gm[q])
        pltpu.async_copy(xp_hbm.at[s_sl], xx[q], gm[q])

    def wait_gather(p, q):
        d_sl = sd[p].at[pl.ds(CHUNK, CHUNK)]
        s_sl = sd[p].at[pl.ds(0, CHUNK)]
        pltpu.make_async_copy(adst_hbm.at[d_sl], bb[q], gm[q]).wait()
        pltpu.make_async_copy(xp_hbm.at[s_sl], xx[q], gm[q]).wait()

    def issue_scatter(p, q):
        d_sl = sd[p].at[pl.ds(CHUNK, CHUNK)]
        pltpu.async_copy(xx[q], numer_sh.at[d_sl], sc[q], add=True)

    def wait_scatter(p, q):
        d_sl = sd[p].at[pl.ds(CHUNK, CHUNK)]
        pltpu.make_async_copy(xx[q], numer_sh.at[d_sl], sc[q]).wait()

    def compute(q):
        b_ref = bb[q]
        x_ref = xx[q]

        @pl.loop(0, CHUNK, step=2)
        def _(e0):
            for u in range(2):
                e = e0 + u
                x1v = [x_ref[e, pl.ds(HP * h + 16, 16)] for h in range(HEADS)]
                al = b_ref[e, :]
                for h in range(HEADS):
                    al = al + x1v[h][15] * oh[h]
                exv = jnp.exp(jnp.maximum(al, 0.2 * al))
                for h in range(HEADS):
                    coef = exv[h]
                    sl0 = pl.ds(HP * h, 16)
                    x_ref[e, sl0] = x_ref[e, sl0] * coef
                    x_ref[e, pl.ds(HP * h + 16, 16)] = (x1v[h] + oh14) * coef

    issue_sd(0, 0)
    issue_sd(1, 1)
    wait_sd(0, 0)
    issue_gather(0, 0)

    @pl.loop(0, STEPS // 6)
    def _(i):
        g0 = 6 * i
        for k in range(6):
            g = g0 + k
            p = k % 3
            pn = (k + 1) % 3
            pf = (k + 2) % 3
            q = k % 2
            qo = 1 - q

            @pl.when(g >= 1)
            def _():
                wait_scatter((k - 1) % 3, qo)

            @pl.when(g + 2 < STEPS)
            def _():
                issue_sd(g + 2, pf)

            @pl.when(g + 1 < STEPS)
            def _():
                wait_sd(g + 1, pn)
                issue_gather(pn, qo)

            wait_gather(p, q)
            compute(q)
            issue_scatter(p, q)

    wait_scatter((STEPS - 1) % 3, (STEPS - 1) % 2)
    plsc.subcore_barrier()

    @pl.loop(0, nz)
    def _(k):
        pltpu.sync_copy(numer_sh.at[pl.ds(row0 + CHUNK * k, CHUNK)],
                        numer_hbm.at[c, pl.ds(row0 + CHUNK * k, CHUNK)])

    pltpu.sync_copy(numer_sh.at[pl.ds(row0 + nz * CHUNK, rem)],
                    numer_hbm.at[c, pl.ds(row0 + nz * CHUNK, rem)])


_EDGE_CALL = pl.kernel(
    _edge_kernel,
    out_type=jax.ShapeDtypeStruct((2, NP, DP), jnp.float32),
    mesh=plsc.VectorSubcoreMesh(core_axis_name="c", subcore_axis_name="s"),
    scratch_types=[
        pltpu.VMEM((2 * CHUNK,), jnp.int32),
        pltpu.VMEM((2 * CHUNK,), jnp.int32),
        pltpu.VMEM((2 * CHUNK,), jnp.int32),
        pltpu.VMEM((CHUNK, 16), jnp.float32),
        pltpu.VMEM((CHUNK, 16), jnp.float32),
        pltpu.VMEM((CHUNK, DP), jnp.float32),
        pltpu.VMEM((CHUNK, DP), jnp.float32),
        pltpu.SemaphoreType.DMA,
        pltpu.SemaphoreType.DMA,
        pltpu.SemaphoreType.DMA,
        pltpu.SemaphoreType.DMA,
        pltpu.SemaphoreType.DMA,
        pltpu.SemaphoreType.DMA,
        pltpu.SemaphoreType.DMA,
        pltpu.VMEM_SHARED((NP, DP), jnp.float32),
    ],
    compiler_params=pltpu.CompilerParams(use_tc_tiling_on_sc=False),
)


def _ext_weight(W, att_src, att_dst):
    fin = W.shape[0]
    Wr = W.reshape(fin, HEADS, C)
    vs = jnp.einsum("fhc,hc->fh", Wr, att_src)
    vd = jnp.einsum("fhc,hc->fh", Wr, att_dst)
    Wh = jnp.concatenate(
        [Wr, jnp.zeros((fin, HEADS, 1), jnp.float32), vs[:, :, None]],
        axis=2).reshape(fin, DP)
    return jnp.concatenate(
        [Wh, vd, jnp.zeros((fin, 10), jnp.float32)], axis=1)


def kernel(x, edge_index, W1, att_src1, att_dst1, b1, W2, att_src2, att_dst2,
           b2, Wl, bl):
    src = edge_index[0]
    dst = edge_index[1]
    loop = jnp.arange(N, dtype=src.dtype)
    pad = jnp.full((E_PAD - E - N,), N, src.dtype)
    s_idx = jnp.concatenate([src, loop, pad]).reshape(-1, CHUNK)
    d_idx = jnp.concatenate([dst, loop, pad]).reshape(-1, CHUNK)
    sd_idx = jnp.stack([s_idx, d_idx], axis=1).reshape(-1)

    b1p = jnp.pad(b1.reshape(HEADS, C), ((0, 0), (0, HP - C))).reshape(1, DP)
    b2p = jnp.pad(b2.reshape(HEADS, C), ((0, 0), (0, HP - C))).reshape(1, DP)

    W1e = _ext_weight(W1, att_src1, att_dst1)
    xp1, adst1 = _proj(x, W1e)
    numer1 = _EDGE_CALL(sd_idx, adst1, xp1)
    h1p = _epilogue(numer1, b1p, elu=True)

    W2e = _ext_weight(_pad_heads(W2), att_src2, att_dst2)
    xp2, adst2 = _proj(h1p, W2e)
    numer2 = _EDGE_CALL(sd_idx, adst2, xp2)
    hp = _epilogue(numer2, b2p, elu=False)

    h = hp.reshape(N, HEADS, HP)[:, :, :C].reshape(N, HEADS * C)
    Wlp = jnp.pad(_pad_heads(Wl), ((0, 0), (0, 118)))
    out = _matmul(hp, Wlp)[:, :10] + bl
    return (out, h)


# speedup vs baseline: 66.3831x; 1.0483x over previous
import jax
import jax.numpy as jnp
from jax import lax
from jax.experimental import pallas as pl
from jax.experimental.pallas import tpu as pltpu
from jax.experimental.pallas import tpu_sc as plsc

N = 10000
E = 320000
HEADS = 6
C = 30
HP = 32
DP = HEADS * HP
NP = 10008
STRIPE = 624
CHUNK = 16
STEPS = 648
E_PAD = 32 * CHUNK * STEPS


def _matmul_kernel(x_ref, w_ref, o_ref):
    o_ref[...] = jnp.dot(x_ref[...], w_ref[...],
                         preferred_element_type=jnp.float32)


def _proj_kernel(x_ref, w_ref, o1_ref, o2_ref):
    r = jnp.dot(x_ref[...], w_ref[...], preferred_element_type=jnp.float32)
    o1_ref[...] = r[:, :DP]
    o2_ref[...] = r[:, DP:]


def _proj(x, w, block_rows=1000):
    n, k = x.shape
    return pl.pallas_call(
        _proj_kernel,
        grid=(n // block_rows,),
        in_specs=[
            pl.BlockSpec((block_rows, k), lambda i: (i, 0)),
            pl.BlockSpec((k, DP + 16), lambda i: (0, 0)),
        ],
        out_specs=[pl.BlockSpec((block_rows, DP), lambda i: (i, 0)),
                   pl.BlockSpec((block_rows, 16), lambda i: (i, 0))],
        out_shape=[jax.ShapeDtypeStruct((NP, DP), jnp.float32),
                   jax.ShapeDtypeStruct((NP, 16), jnp.float32)],
    )(x, w)


def _epi_kernel_elu(nu_ref, b_ref, o_ref):
    _epi_body(nu_ref, b_ref, o_ref, True)


def _epi_kernel_lin(nu_ref, b_ref, o_ref):
    _epi_body(nu_ref, b_ref, o_ref, False)


def _epi_body(nu_ref, b_ref, o_ref, elu):
    acc = nu_ref[0] + nu_ref[1]
    for h in range(HEADS):
        a = acc[:, HP * h:HP * h + C]
        den = acc[:, HP * h + C:HP * h + C + 1]
        v = a / (den + 1e-16) + b_ref[:, HP * h:HP * h + C]
        if elu:
            v = jnp.where(v > 0, v, jnp.exp(v) - 1.0)
        o_ref[:, HP * h:HP * h + C] = v
        o_ref[:, HP * h + C:HP * h + HP] = jnp.zeros(
            (o_ref.shape[0], HP - C), jnp.float32)


def _epilogue(numer, bias_pad, elu, block_rows=1000):
    body = _epi_kernel_elu if elu else _epi_kernel_lin
    return pl.pallas_call(
        body,
        grid=(N // block_rows,),
        in_specs=[
            pl.BlockSpec((2, block_rows, DP), lambda i: (0, i, 0)),
            pl.BlockSpec((1, DP), lambda i: (0, 0)),
        ],
        out_specs=pl.BlockSpec((block_rows, DP), lambda i: (i, 0)),
        out_shape=jax.ShapeDtypeStruct((N, DP), jnp.float32),
    )(numer, bias_pad)


def _pad_heads(a):
    return jnp.pad(a.reshape(HEADS, C, -1),
                   ((0, 0), (0, HP - C), (0, 0))).reshape(HEADS * HP, -1)


def _matmul(x, w, block_rows=1000):
    n, k = x.shape
    m = w.shape[1]
    return pl.pallas_call(
        _matmul_kernel,
        grid=(n // block_rows,),
        in_specs=[
            pl.BlockSpec((block_rows, k), lambda i: (i, 0)),
            pl.BlockSpec((k, m), lambda i: (0, 0)),
        ],
        out_specs=pl.BlockSpec((block_rows, m), lambda i: (i, 0)),
        out_shape=jax.ShapeDtypeStruct((n, m), jnp.float32),
    )(x, w)


def _edge_kernel(sd_hbm, adst_hbm, xp_hbm,
                 numer_hbm,
                 sd0, sd1, sd2, sd3, sd4, sd5, b0, b1, b2, x0, x1, x2,
                 sdm0, sdm1, sdm2, sdm3, sdm4, sdm5,
                 gm0, gm1, gm2, sc0, sc1, sc2,
                 numer_sh):
    c = lax.axis_index("c")
    t = lax.axis_index("s")
    w = c * 16 + t
    row0 = t * STRIPE

    sd = [sd0, sd1, sd2, sd3, sd4, sd5]
    sdm = [sdm0, sdm1, sdm2, sdm3, sdm4, sdm5]
    bb = [b0, b1, b2]
    xx = [x0, x1, x2]
    gm = [gm0, gm1, gm2]
    sc = [sc0, sc1, sc2]

    lane = lax.iota(jnp.int32, 16)
    z16 = jnp.where(lane < 0, 1.0, 0.0).astype(jnp.float32)
    oh = [jnp.where(lane == i, 1.0, 0.0).astype(jnp.float32)
          for i in range(HEADS)]
    oh14 = jnp.where(lane == 14, 1.0, 0.0).astype(jnp.float32)

    @pl.loop(0, CHUNK)
    def _(i):
        for j in range(DP // 16):
            x0[i, pl.ds(16 * j, 16)] = z16

    nz = STRIPE // CHUNK

    @pl.loop(0, nz)
    def _(k):
        pltpu.sync_copy(x0, numer_sh.at[pl.ds(row0 + CHUNK * k, CHUNK)])

    @pl.when(t == 15)
    def _():
        pltpu.sync_copy(x0, numer_sh.at[pl.ds(row0 + nz * CHUNK, CHUNK)])
        pltpu.sync_copy(x0.at[pl.ds(0, 8)],
                        numer_sh.at[pl.ds(row0 + nz * CHUNK + CHUNK, 8)])
    plsc.subcore_barrier()

    base0 = w * STEPS

    def sd_src(g):
        return sd_hbm.at[pl.ds((base0 + g) * 2 * CHUNK, 2 * CHUNK)]

    def issue_sd(g, p):
        pltpu.async_copy(sd_src(g), sd[p], sdm[p])

    def wait_sd(g, p):
        pltpu.make_async_copy(sd_src(g), sd[p], sdm[p]).wait()

    def issue_gather(p, q):
        s_sl = sd[p].at[pl.ds(0, CHUNK)]
        d_sl = sd[p].at[pl.ds(CHUNK, CHUNK)]
        pltpu.async_copy(adst_hbm.at[d_sl], bb[q], gm[q])
        pltpu.async_copy(xp_hbm.at[s_sl], xx[q], gm[q])

    def wait_gather(p, q):
        d_sl = sd[p].at[pl.ds(CHUNK, CHUNK)]
        s_sl = sd[p].at[pl.ds(0, CHUNK)]
        pltpu.make_async_copy(adst_hbm.at[d_sl], bb[q], gm[q]).wait()
        pltpu.make_async_copy(xp_hbm.at[s_sl], xx[q], gm[q]).wait()

    def issue_scatter(p, q):
        d_sl = sd[p].at[pl.ds(CHUNK, CHUNK)]
        pltpu.async_copy(xx[q], numer_sh.at[d_sl], sc[q], add=True)

    def wait_scatter(p, q):
        d_sl = sd[p].at[pl.ds(CHUNK, CHUNK)]
        pltpu.make_async_copy(xx[q], numer_sh.at[d_sl], sc[q]).wait()

    def compute(q):
        b_ref = bb[q]
        x_ref = xx[q]

        @pl.loop(0, CHUNK, step=2)
        def _(e0):
            for u in range(2):
                e = e0 + u
                x1v = [x_ref[e, pl.ds(HP * h + 16, 16)] for h in range(HEADS)]
                al = b_ref[e, :]
                for h in range(HEADS):
                    al = al + x1v[h][15] * oh[h]
                exv = jnp.exp(jnp.maximum(al, 0.2 * al))
                for h in range(HEADS):
                    coef = exv[h]
                    sl0 = pl.ds(HP * h, 16)
                    x_ref[e, sl0] = x_ref[e, sl0] * coef
                    x_ref[e, pl.ds(HP * h + 16, 16)] = (x1v[h] + oh14) * coef

    for g in range(4):
        issue_sd(g, g)
    wait_sd(0, 0)
    issue_gather(0, 0)
    wait_sd(1, 1)
    issue_gather(1, 1)

    @pl.loop(0, STEPS // 6)
    def _(i):
        g0 = 6 * i
        for k in range(6):
            g = g0 + k
            q = k % 3
            p = k % 6

            @pl.when(g >= 1)
            def _():
                wait_scatter((p + 5) % 6, (k + 2) % 3)

            @pl.when(g + 2 < STEPS)
            def _():
                wait_sd(g + 2, (p + 2) % 6)
                issue_gather((p + 2) % 6, (k + 2) % 3)

            wait_gather(p, q)
            compute(q)
            issue_scatter(p, q)

            @pl.when(g + 4 < STEPS)
            def _():
                issue_sd(g + 4, (p + 4) % 6)

    wait_scatter((STEPS - 1) % 6, (STEPS - 1) % 3)
    plsc.subcore_barrier()

    @pl.loop(0, nz)
    def _(k):
        pltpu.sync_copy(numer_sh.at[pl.ds(row0 + CHUNK * k, CHUNK)],
                        numer_hbm.at[c, pl.ds(row0 + CHUNK * k, CHUNK)])

    @pl.when(t == 15)
    def _():
        pltpu.sync_copy(numer_sh.at[pl.ds(row0 + nz * CHUNK, CHUNK)],
                        numer_hbm.at[c, pl.ds(row0 + nz * CHUNK, CHUNK)])
        pltpu.sync_copy(numer_sh.at[pl.ds(row0 + nz * CHUNK + CHUNK, 8)],
                        numer_hbm.at[c, pl.ds(row0 + nz * CHUNK + CHUNK, 8)])


_EDGE_CALL = pl.kernel(
    _edge_kernel,
    out_type=jax.ShapeDtypeStruct((2, NP, DP), jnp.float32),
    mesh=plsc.VectorSubcoreMesh(core_axis_name="c", subcore_axis_name="s"),
    scratch_types=(
        [pltpu.VMEM((2 * CHUNK,), jnp.int32)] * 6
        + [pltpu.VMEM((CHUNK, 16), jnp.float32)] * 3
        + [pltpu.VMEM((CHUNK, DP), jnp.float32)] * 3
        + [pltpu.SemaphoreType.DMA] * 12
        + [pltpu.VMEM_SHARED((NP, DP), jnp.float32)]
    ),
    compiler_params=pltpu.CompilerParams(use_tc_tiling_on_sc=False),
)


def _ext_weight(W, att_src, att_dst):
    fin = W.shape[0]
    Wr = W.reshape(fin, HEADS, C)
    vs = jnp.einsum("fhc,hc->fh", Wr, att_src)
    vd = jnp.einsum("fhc,hc->fh", Wr, att_dst)
    Wh = jnp.concatenate(
        [Wr, jnp.zeros((fin, HEADS, 1), jnp.float32), vs[:, :, None]],
        axis=2).reshape(fin, DP)
    return jnp.concatenate(
        [Wh, vd, jnp.zeros((fin, 10), jnp.float32)], axis=1)


def kernel(x, edge_index, W1, att_src1, att_dst1, b1, W2, att_src2, att_dst2,
           b2, Wl, bl):
    src = edge_index[0]
    dst = edge_index[1]
    loop = jnp.arange(N, dtype=src.dtype)
    pad = jnp.full((E_PAD - E - N,), N, src.dtype)
    s_idx = jnp.concatenate([src, loop, pad]).reshape(-1, CHUNK)
    d_idx = jnp.concatenate([dst, loop, pad]).reshape(-1, CHUNK)
    sd_idx = jnp.stack([s_idx, d_idx], axis=1).reshape(-1)

    b1p = jnp.pad(b1.reshape(HEADS, C), ((0, 0), (0, HP - C))).reshape(1, DP)
    b2p = jnp.pad(b2.reshape(HEADS, C), ((0, 0), (0, HP - C))).reshape(1, DP)

    W1e = _ext_weight(W1, att_src1, att_dst1)
    xp1, adst1 = _proj(x, W1e)
    numer1 = _EDGE_CALL(sd_idx, adst1, xp1)
    h1p = _epilogue(numer1, b1p, elu=True)

    W2e = _ext_weight(_pad_heads(W2), att_src2, att_dst2)
    xp2, adst2 = _proj(h1p, W2e)
    numer2 = _EDGE_CALL(sd_idx, adst2, xp2)
    hp = _epilogue(numer2, b2p, elu=False)

    h = hp.reshape(N, HEADS, HP)[:, :, :C].reshape(N, HEADS * C)
    Wlp = jnp.pad(_pad_heads(Wl), ((0, 0), (0, 118)))
    out = _matmul(hp, Wlp)[:, :10] + bl
    return (out, h)
